# Initial kernel scaffold; baseline (speedup 1.0000x reference)
import jax
import jax.numpy as jnp
from jax.experimental import pallas as pl


def kernel(feats, edge_index, etype, norm, bases0, coeff0, bias0, bases1, coeff1, bias1, bases2, coeff2, bias2):
    raise NotImplementedError("write your pallas kernel here")


# speedup vs baseline: 2.2909x; 2.2909x over previous
import functools

import jax
import jax.numpy as jnp
from jax.experimental import pallas as pl
from jax.experimental.pallas import tpu as pltpu

N_NODES = 10000
N_EDGES = 320000
NUM_RELS = 8


def _mm_body(x_ref, w_ref, o_ref):
    o_ref[...] = jnp.dot(x_ref[...], w_ref[...],
                         preferred_element_type=jnp.float32)


def _mm(x, w, block_rows=1000):
    n, k = x.shape
    k2, m = w.shape
    grid = n // block_rows
    return pl.pallas_call(
        _mm_body,
        grid=(grid,),
        in_specs=[
            pl.BlockSpec((block_rows, k), lambda i: (i, 0)),
            pl.BlockSpec((k, m), lambda i: (0, 0)),
        ],
        out_specs=pl.BlockSpec((block_rows, m), lambda i: (i, 0)),
        out_shape=jax.ShapeDtypeStruct((n, m), jnp.float32),
    )(x, w)


def _layer(h, eidx, dst, norm, bases, coeff, bias, relu):
    din = bases.shape[1]
    dout = bases.shape[2]
    w = jnp.einsum("rb,bio->iro", coeff, bases).reshape(din, NUM_RELS * dout)
    z = _mm(h, w).reshape(N_NODES * NUM_RELS, dout)
    msg = z[eidx] * norm
    out = jax.ops.segment_sum(msg, dst, num_segments=N_NODES) + bias
    if relu:
        out = jax.nn.relu(out)
    return out


def kernel(feats, edge_index, etype, norm, bases0, coeff0, bias0,
           bases1, coeff1, bias1, bases2, coeff2, bias2):
    src = edge_index[0].astype(jnp.int32)
    dst = edge_index[1].astype(jnp.int32)
    et = etype.astype(jnp.int32)
    eidx = src * NUM_RELS + et
    h = _layer(feats, eidx, dst, norm, bases0, coeff0, bias0, True)
    h = _layer(h, eidx, dst, norm, bases1, coeff1, bias1, True)
    h = _layer(h, eidx, dst, norm, bases2, coeff2, bias2, False)
    return h


# speedup vs baseline: 8.3272x; 3.6348x over previous
import functools

import jax
import jax.numpy as jnp
from jax import lax
from jax.experimental import pallas as pl
from jax.experimental.pallas import tpu as pltpu
from jax.experimental.pallas import tpu_sc as plsc

N_NODES = 10000
NP = 10240
N_EDGES = 320000
NUM_RELS = 8
K = 128
NW = 32
N_CHUNKS = N_EDGES // K


def _mm0_body(x_ref, w_ref, o_ref):
    o_ref[...] = jnp.dot(x_ref[...], w_ref[...],
                         preferred_element_type=jnp.float32)


def _mm0(x, w, block_rows=1000):
    n, k = x.shape
    _, m = w.shape
    return pl.pallas_call(
        _mm0_body,
        grid=(n // block_rows,),
        in_specs=[
            pl.BlockSpec((block_rows, k), lambda i: (i, 0)),
            pl.BlockSpec((k, m), lambda i: (0, 0)),
        ],
        out_specs=pl.BlockSpec((block_rows, m), lambda i: (i, 0)),
        out_shape=jax.ShapeDtypeStruct((n, m), jnp.float32),
    )(x, w)


def _mm_fused_body(p_ref, b_ref, w_ref, o_ref):
    x = jax.nn.relu(p_ref[0] + p_ref[1] + b_ref[...])
    o_ref[...] = jnp.dot(x, w_ref[...], preferred_element_type=jnp.float32)


def _mm_fused(partials, bias, w, block_rows=1024):
    _, n, k = partials.shape
    _, m = w.shape
    return pl.pallas_call(
        _mm_fused_body,
        grid=(n // block_rows,),
        in_specs=[
            pl.BlockSpec((2, block_rows, k), lambda i: (0, i, 0)),
            pl.BlockSpec((1, k), lambda i: (0, 0)),
        ] + [pl.BlockSpec((k, m), lambda i: (0, 0))],
        out_specs=pl.BlockSpec((block_rows, m), lambda i: (i, 0)),
        out_shape=jax.ShapeDtypeStruct((n, m), jnp.float32),
    )(partials, bias.reshape(1, k), w)


def _combine_body(p_ref, b_ref, o_ref):
    acc = b_ref[...]
    s = p_ref[0] + p_ref[1]
    for d in range(8):
        acc = acc + s[:, d * 16:(d + 1) * 16]
    o_ref[...] = acc


def _combine(partials, bias, n_out, block_rows=1000):
    _, n, k = partials.shape
    m = bias.shape[0]
    return pl.pallas_call(
        _combine_body,
        grid=(n_out // block_rows,),
        in_specs=[
            pl.BlockSpec((2, block_rows, k), lambda i: (0, i, 0)),
            pl.BlockSpec((1, m), lambda i: (0, 0)),
        ],
        out_specs=pl.BlockSpec((block_rows, m), lambda i: (i, 0)),
        out_shape=jax.ShapeDtypeStruct((n_out, m), jnp.float32),
    )(partials, bias.reshape(1, m))


def _sc_layer(src, et, dst, norm, z, dout, block_select=False):
    mesh = plsc.VectorSubcoreMesh(core_axis_name="c", subcore_axis_name="s")
    rows_per_tile = NP // 16

    @functools.partial(
        pl.kernel,
        out_type=jax.ShapeDtypeStruct((2, NP, dout), jnp.float32),
        mesh=mesh,
        scratch_types=[
            pltpu.VMEM((K,), jnp.int32),
            pltpu.VMEM((K,), jnp.int32),
            pltpu.VMEM((K,), jnp.int32),
            pltpu.VMEM((K,), jnp.float32),
            pltpu.VMEM((K,), jnp.int32),
            pltpu.VMEM((K, dout), jnp.float32),
            pltpu.VMEM_SHARED((NP, dout), jnp.float32),
            pltpu.SemaphoreType.DMA,
        ],
    )
    def sc_kernel(src_h, et_h, dst_h, norm_h, z_h, out_h,
                  src_v, et_v, dst_v, norm_v, idx_v, rows_v, accum, sem):
        c = lax.axis_index("c")
        s = lax.axis_index("s")
        wid = s * 2 + c

        def zrow(r, _):
            for d in range(dout // 16):
                rows_v[r, pl.ds(d * 16, 16)] = jnp.zeros((16,), jnp.float32)
            return 0
        lax.fori_loop(0, K, zrow, 0)
        for j in range(rows_per_tile // K):
            pltpu.sync_copy(rows_v, accum.at[pl.ds(s * rows_per_tile + j * K, K)])
        plsc.subcore_barrier()

        n_chunks = 78 + jnp.where(wid < N_CHUNKS - 78 * NW, 1, 0)

        def chunk(i, _):
            base = (wid + i * NW) * K
            pltpu.sync_copy(src_h.at[pl.ds(base, K)], src_v)
            pltpu.sync_copy(et_h.at[pl.ds(base, K)], et_v)
            pltpu.sync_copy(dst_h.at[pl.ds(base, K)], dst_v)
            pltpu.sync_copy(norm_h.at[pl.ds(base, K)], norm_v)
            for j in range(K // 16):
                sl = pl.ds(j * 16, 16)
                if block_select:
                    idx_v[sl] = src_v[sl]
                else:
                    idx_v[sl] = src_v[sl] * NUM_RELS + et_v[sl]
            pltpu.async_copy(z_h.at[idx_v], rows_v, sem).wait()

            def scale(g, _):
                nv = norm_v[pl.ds(g * 16, 16)]
                if block_select:
                    ev = et_v[pl.ds(g * 16, 16)]
                for l in range(16):
                    e = g * 16 + l
                    n_e = nv[l]
                    for d in range(dout // 16):
                        sl = pl.ds(d * 16, 16)
                        if block_select:
                            f = jnp.where(ev[l] == d, n_e, 0.0)
                        else:
                            f = n_e
                        rows_v[e, sl] = rows_v[e, sl] * f
                return 0
            lax.fori_loop(0, K // 16, scale, 0)
            pltpu.sync_copy(rows_v, accum.at[dst_v], add=True)
            return 0

        lax.fori_loop(0, n_chunks, chunk, 0)
        plsc.subcore_barrier()
        pltpu.sync_copy(accum.at[pl.ds(s * rows_per_tile, rows_per_tile)],
                        out_h.at[c, pl.ds(s * rows_per_tile, rows_per_tile)])

    return sc_kernel(src, et, dst, norm, z)


def _wcat(bases, coeff):
    din, dout = bases.shape[1], bases.shape[2]
    return jnp.einsum("rb,bio->iro", coeff, bases).reshape(din, NUM_RELS * dout)


def kernel(feats, edge_index, etype, norm, bases0, coeff0, bias0,
           bases1, coeff1, bias1, bases2, coeff2, bias2):
    src = edge_index[0].astype(jnp.int32)
    dst = edge_index[1].astype(jnp.int32)
    et = etype.astype(jnp.int32)
    nrm = norm.reshape(-1)

    feats_p = jnp.pad(feats, ((0, NP - N_NODES), (0, 0)))

    z0 = _mm0(feats_p, _wcat(bases0, coeff0), block_rows=1024)
    p0 = _sc_layer(src, et, dst, nrm, z0.reshape(NP * NUM_RELS, 128), 128)
    z1 = _mm_fused(p0, bias0, _wcat(bases1, coeff1))
    p1 = _sc_layer(src, et, dst, nrm, z1.reshape(NP * NUM_RELS, 128), 128)
    z2 = _mm_fused(p1, bias1, _wcat(bases2, coeff2))
    p2 = _sc_layer(src, et, dst, nrm, z2, 128, block_select=True)
    out = _combine(p2, bias2, N_NODES, block_rows=1000)
    return out


# speedup vs baseline: 16.0509x; 1.9275x over previous
import functools

import jax
import jax.numpy as jnp
from jax import lax
from jax.experimental import pallas as pl
from jax.experimental.pallas import tpu as pltpu
from jax.experimental.pallas import tpu_sc as plsc

N_NODES = 10000
NP = 10240
N_EDGES = 320000
NUM_RELS = 8
K = 80
NW = 32
N_CHUNKS = N_EDGES // K
CHUNKS_PER_TILE = N_CHUNKS // NW


def _mm0_body(x_ref, w_ref, o_ref):
    o_ref[...] = jnp.dot(x_ref[...], w_ref[...],
                         preferred_element_type=jnp.float32)


def _mm0(x, w, block_rows=1024):
    n, k = x.shape
    _, m = w.shape
    return pl.pallas_call(
        _mm0_body,
        grid=(n // block_rows,),
        in_specs=[
            pl.BlockSpec((block_rows, k), lambda i: (i, 0)),
            pl.BlockSpec((k, m), lambda i: (0, 0)),
        ],
        out_specs=pl.BlockSpec((block_rows, m), lambda i: (i, 0)),
        out_shape=jax.ShapeDtypeStruct((n, m), jnp.float32),
    )(x, w)


def _mm_fused_body(p_ref, b_ref, w_ref, o_ref):
    x = jax.nn.relu(p_ref[0] + p_ref[1] + b_ref[...])
    o_ref[...] = jnp.dot(x, w_ref[...], preferred_element_type=jnp.float32)


def _mm_fused(partials, bias, w, block_rows=1024):
    _, n, k = partials.shape
    _, m = w.shape
    return pl.pallas_call(
        _mm_fused_body,
        grid=(n // block_rows,),
        in_specs=[
            pl.BlockSpec((2, block_rows, k), lambda i: (0, i, 0)),
            pl.BlockSpec((1, k), lambda i: (0, 0)),
            pl.BlockSpec((k, m), lambda i: (0, 0)),
        ],
        out_specs=pl.BlockSpec((block_rows, m), lambda i: (i, 0)),
        out_shape=jax.ShapeDtypeStruct((n, m), jnp.float32),
    )(partials, bias.reshape(1, k), w)


def _combine_body(p_ref, b_ref, o_ref):
    acc = b_ref[...]
    s = p_ref[0] + p_ref[1]
    for d in range(8):
        acc = acc + s[:, d * 16:(d + 1) * 16]
    o_ref[...] = acc


def _combine(partials, bias, n_out, block_rows=1000):
    _, n, k = partials.shape
    m = bias.shape[0]
    return pl.pallas_call(
        _combine_body,
        grid=(n_out // block_rows,),
        in_specs=[
            pl.BlockSpec((2, block_rows, k), lambda i: (0, i, 0)),
            pl.BlockSpec((1, m), lambda i: (0, 0)),
        ],
        out_specs=pl.BlockSpec((block_rows, m), lambda i: (i, 0)),
        out_shape=jax.ShapeDtypeStruct((n_out, m), jnp.float32),
    )(partials, bias.reshape(1, m))


NBUF = 3


def _sc_layer(edata, z, dout, block_select=False):
    mesh = plsc.VectorSubcoreMesh(core_axis_name="c", subcore_axis_name="s")
    rows_per_tile = NP // 16

    @functools.partial(
        pl.kernel,
        out_type=jax.ShapeDtypeStruct((2, NP, dout), jnp.float32),
        mesh=mesh,
        scratch_types=(
            [pltpu.VMEM((4, K), jnp.int32) for _ in range(NBUF)]
            + [pltpu.VMEM((K,), jnp.int32) for _ in range(NBUF)]
            + [pltpu.VMEM((K, dout), jnp.float32) for _ in range(NBUF)]
            + [pltpu.VMEM_SHARED((NP, dout), jnp.float32)]
            + [pltpu.SemaphoreType.DMA for _ in range(2 * NBUF)]
        ),
    )
    def sc_kernel(edata_h, z_h, out_h, *refs):
        ebuf = refs[0:NBUF]
        idxb = refs[NBUF:2 * NBUF]
        rows = refs[2 * NBUF:3 * NBUF]
        accum = refs[3 * NBUF]
        gsem = refs[3 * NBUF + 1:4 * NBUF + 1]
        ssem = refs[4 * NBUF + 1:5 * NBUF + 1]

        c = lax.axis_index("c")
        s = lax.axis_index("s")
        wid = s * 2 + c

        def zrow(r, _):
            for d in range(dout // 16):
                rows[0][r, pl.ds(d * 16, 16)] = jnp.zeros((16,), jnp.float32)
            return 0
        lax.fori_loop(0, K, zrow, 0)
        for j in range(rows_per_tile // K):
            pltpu.sync_copy(rows[0],
                            accum.at[pl.ds(s * rows_per_tile + j * K, K)])
        plsc.subcore_barrier()

        def load_chunk(i, p):
            pltpu.sync_copy(edata_h.at[wid + i * NW], ebuf[p])
            for j in range(K // 16):
                sl = pl.ds(j * 16, 16)
                if block_select:
                    idxb[p][sl] = ebuf[p][0, sl]
                else:
                    idxb[p][sl] = ebuf[p][0, sl] * NUM_RELS + ebuf[p][1, sl]

        def issue_gather(p):
            pltpu.async_copy(z_h.at[idxb[p]], rows[p], gsem[p])

        def wait_gather(p):
            pltpu.make_async_copy(z_h.at[idxb[p]], rows[p], gsem[p]).wait()

        def issue_scatter(p):
            pltpu.async_copy(rows[p], accum.at[ebuf[p].at[2]], ssem[p],
                             add=True)

        def wait_scatter(p):
            pltpu.make_async_copy(rows[p], accum.at[ebuf[p].at[2]],
                                  ssem[p]).wait()

        def scale_rows(p):
            def scale(g, _):
                nv = lax.bitcast_convert_type(ebuf[p][3, pl.ds(g * 16, 16)],
                                              jnp.float32)
                if block_select:
                    ev = ebuf[p][1, pl.ds(g * 16, 16)]
                for l in range(16):
                    e = g * 16 + l
                    n_e = nv[l]
                    for d in range(dout // 16):
                        sl = pl.ds(d * 16, 16)
                        if block_select:
                            f = jnp.where(ev[l] == d, n_e, 0.0)
                        else:
                            f = n_e
                        rows[p][e, sl] = rows[p][e, sl] * f
                return 0
            lax.fori_loop(0, K // 16, scale, 0)

        def pre(i, p, first):
            if not first:
                wait_scatter(p)
            load_chunk(i, p)
            issue_gather(p)

        def post(i, p):
            wait_gather(p)
            scale_rows(p)
            issue_scatter(p)

        n = CHUNKS_PER_TILE
        n_steps = (n - 2) // NBUF

        def superstep(si, _, first_sup=False):
            cbase = si * NBUF
            for b in range(NBUF):
                post(cbase + b, b)
                pre(cbase + b + 2, (b + 2) % NBUF, first_sup and b == 0)
            return 0

        pre(0, 0, True)
        pre(1, 1, True)
        superstep(0, 0, first_sup=True)
        lax.fori_loop(1, n_steps, superstep, 0)

        for t in range(n - NBUF * n_steps):
            ct = NBUF * n_steps + t
            post(ct, ct % NBUF)
            if ct + 2 < n:
                pre(ct + 2, (ct + 2) % NBUF, False)
        for ct in range(n - NBUF, n):
            wait_scatter(ct % NBUF)

        plsc.subcore_barrier()
        pltpu.sync_copy(accum.at[pl.ds(s * rows_per_tile, rows_per_tile)],
                        out_h.at[c, pl.ds(s * rows_per_tile, rows_per_tile)])

    return sc_kernel(edata, z)


def _wcat(bases, coeff):
    din, dout = bases.shape[1], bases.shape[2]
    return jnp.einsum("rb,bio->iro", coeff, bases).reshape(din, NUM_RELS * dout)


def kernel(feats, edge_index, etype, norm, bases0, coeff0, bias0,
           bases1, coeff1, bias1, bases2, coeff2, bias2):
    src = edge_index[0].astype(jnp.int32)
    dst = edge_index[1].astype(jnp.int32)
    et = etype.astype(jnp.int32)
    nbits = lax.bitcast_convert_type(norm.reshape(-1), jnp.int32)

    edata = jnp.stack([src, et, dst, nbits], axis=0)
    edata = edata.reshape(4, N_CHUNKS, K).transpose(1, 0, 2)

    feats_p = jnp.pad(feats, ((0, NP - N_NODES), (0, 0)))

    z0 = _mm0(feats_p, _wcat(bases0, coeff0))
    p0 = _sc_layer(edata, z0.reshape(NP * NUM_RELS, 128), 128)
    z1 = _mm_fused(p0, bias0, _wcat(bases1, coeff1))
    p1 = _sc_layer(edata, z1.reshape(NP * NUM_RELS, 128), 128)
    z2 = _mm_fused(p1, bias1, _wcat(bases2, coeff2))
    p2 = _sc_layer(edata, z2, 128, block_select=True)
    out = _combine(p2, bias2, N_NODES, block_rows=1000)
    return out


# speedup vs baseline: 16.2110x; 1.0100x over previous
import functools

import jax
import jax.numpy as jnp
from jax import lax
from jax.experimental import pallas as pl
from jax.experimental.pallas import tpu as pltpu
from jax.experimental.pallas import tpu_sc as plsc

N_NODES = 10000
NP = 10240
N_EDGES = 320000
NUM_RELS = 8
K = 80
NW = 32
N_CHUNKS = N_EDGES // K
CHUNKS_PER_TILE = N_CHUNKS // NW


def _mm0_body(x_ref, w_ref, o_ref):
    o_ref[...] = jnp.dot(x_ref[...], w_ref[...],
                         preferred_element_type=jnp.float32)


def _mm0(x, w, block_rows=1024):
    n, k = x.shape
    _, m = w.shape
    dout = m // NUM_RELS
    nb = n // block_rows
    return pl.pallas_call(
        _mm0_body,
        grid=(nb, NUM_RELS),
        in_specs=[
            pl.BlockSpec((block_rows, k), lambda i, r: (i, 0)),
            pl.BlockSpec((k, dout), lambda i, r: (0, r)),
        ],
        out_specs=pl.BlockSpec((block_rows, dout), lambda i, r: (r * nb + i, 0)),
        out_shape=jax.ShapeDtypeStruct((NUM_RELS * n, dout), jnp.float32),
    )(x, w)


def _mm_fused_body(p_ref, b_ref, w_ref, o_ref):
    x = jax.nn.relu(p_ref[0] + p_ref[1] + b_ref[...])
    o_ref[...] = jnp.dot(x, w_ref[...], preferred_element_type=jnp.float32)


def _mm_fused(partials, bias, w, block_rows=1024, rel_major=True):
    _, n, k = partials.shape
    _, m = w.shape
    if not rel_major:
        return pl.pallas_call(
            _mm_fused_body,
            grid=(n // block_rows,),
            in_specs=[
                pl.BlockSpec((2, block_rows, k), lambda i: (0, i, 0)),
                pl.BlockSpec((1, k), lambda i: (0, 0)),
                pl.BlockSpec((k, m), lambda i: (0, 0)),
            ],
            out_specs=pl.BlockSpec((block_rows, m), lambda i: (i, 0)),
            out_shape=jax.ShapeDtypeStruct((n, m), jnp.float32),
        )(partials, bias.reshape(1, k), w)
    dout = m // NUM_RELS
    nb = n // block_rows
    return pl.pallas_call(
        _mm_fused_body,
        grid=(nb, NUM_RELS),
        in_specs=[
            pl.BlockSpec((2, block_rows, k), lambda i, r: (0, i, 0)),
            pl.BlockSpec((1, k), lambda i, r: (0, 0)),
            pl.BlockSpec((k, dout), lambda i, r: (0, r)),
        ],
        out_specs=pl.BlockSpec((block_rows, dout), lambda i, r: (r * nb + i, 0)),
        out_shape=jax.ShapeDtypeStruct((NUM_RELS * n, dout), jnp.float32),
    )(partials, bias.reshape(1, k), w)


def _combine_body(p_ref, b_ref, o_ref):
    acc = b_ref[...]
    s = p_ref[0] + p_ref[1]
    for d in range(8):
        acc = acc + s[:, d * 16:(d + 1) * 16]
    o_ref[...] = acc


def _combine(partials, bias, n_out, block_rows=1000):
    _, n, k = partials.shape
    m = bias.shape[0]
    return pl.pallas_call(
        _combine_body,
        grid=(n_out // block_rows,),
        in_specs=[
            pl.BlockSpec((2, block_rows, k), lambda i: (0, i, 0)),
            pl.BlockSpec((1, m), lambda i: (0, 0)),
        ],
        out_specs=pl.BlockSpec((block_rows, m), lambda i: (i, 0)),
        out_shape=jax.ShapeDtypeStruct((n_out, m), jnp.float32),
    )(partials, bias.reshape(1, m))


NBUF = 3


def _sc_layer(edata, z, dout, block_select=False):
    mesh = plsc.VectorSubcoreMesh(core_axis_name="c", subcore_axis_name="s")
    rows_per_tile = NP // 16

    @functools.partial(
        pl.kernel,
        out_type=jax.ShapeDtypeStruct((2, NP, dout), jnp.float32),
        mesh=mesh,
        scratch_types=(
            [pltpu.VMEM((4, K), jnp.int32) for _ in range(NBUF)]
            + [pltpu.VMEM((K,), jnp.int32) for _ in range(NBUF)]
            + [pltpu.VMEM((K, dout), jnp.float32) for _ in range(NBUF)]
            + [pltpu.VMEM_SHARED((NP, dout), jnp.float32)]
            + [pltpu.SemaphoreType.DMA for _ in range(2 * NBUF)]
        ),
    )
    def sc_kernel(edata_h, z_h, out_h, *refs):
        ebuf = refs[0:NBUF]
        idxb = refs[NBUF:2 * NBUF]
        rows = refs[2 * NBUF:3 * NBUF]
        accum = refs[3 * NBUF]
        gsem = refs[3 * NBUF + 1:4 * NBUF + 1]
        ssem = refs[4 * NBUF + 1:5 * NBUF + 1]

        c = lax.axis_index("c")
        s = lax.axis_index("s")
        wid = s * 2 + c

        def zrow(r, _):
            for d in range(dout // 16):
                rows[0][r, pl.ds(d * 16, 16)] = jnp.zeros((16,), jnp.float32)
            return 0
        lax.fori_loop(0, K, zrow, 0)
        for j in range(rows_per_tile // K):
            pltpu.sync_copy(rows[0],
                            accum.at[pl.ds(s * rows_per_tile + j * K, K)])
        plsc.subcore_barrier()

        def load_chunk(i, p):
            pltpu.sync_copy(edata_h.at[wid + i * NW], ebuf[p])
            for j in range(K // 16):
                sl = pl.ds(j * 16, 16)
                if block_select:
                    idxb[p][sl] = ebuf[p][0, sl]
                else:
                    idxb[p][sl] = ebuf[p][1, sl] * NP + ebuf[p][0, sl]

        def issue_gather(p):
            pltpu.async_copy(z_h.at[idxb[p]], rows[p], gsem[p])

        def wait_gather(p):
            pltpu.make_async_copy(z_h.at[idxb[p]], rows[p], gsem[p]).wait()

        def issue_scatter(p):
            pltpu.async_copy(rows[p], accum.at[ebuf[p].at[2]], ssem[p],
                             add=True)

        def wait_scatter(p):
            pltpu.make_async_copy(rows[p], accum.at[ebuf[p].at[2]],
                                  ssem[p]).wait()

        def scale_rows(p):
            def scale(g, _):
                nv = lax.bitcast_convert_type(ebuf[p][3, pl.ds(g * 16, 16)],
                                              jnp.float32)
                if block_select:
                    ev = ebuf[p][1, pl.ds(g * 16, 16)]
                for l in range(16):
                    e = g * 16 + l
                    n_e = nv[l]
                    for d in range(dout // 16):
                        sl = pl.ds(d * 16, 16)
                        if block_select:
                            f = jnp.where(ev[l] == d, n_e, 0.0)
                        else:
                            f = n_e
                        rows[p][e, sl] = rows[p][e, sl] * f
                return 0
            lax.fori_loop(0, K // 16, scale, 0)

        def pre(i, p, first):
            if not first:
                wait_scatter(p)
            load_chunk(i, p)
            issue_gather(p)

        def post(i, p):
            wait_gather(p)
            scale_rows(p)
            issue_scatter(p)

        n = CHUNKS_PER_TILE
        n_steps = (n - 2) // NBUF

        def superstep(si, _, first_sup=False):
            cbase = si * NBUF
            for b in range(NBUF):
                post(cbase + b, b)
                pre(cbase + b + 2, (b + 2) % NBUF, first_sup and b == 0)
            return 0

        pre(0, 0, True)
        pre(1, 1, True)
        superstep(0, 0, first_sup=True)
        lax.fori_loop(1, n_steps, superstep, 0)

        for t in range(n - NBUF * n_steps):
            ct = NBUF * n_steps + t
            post(ct, ct % NBUF)
            if ct + 2 < n:
                pre(ct + 2, (ct + 2) % NBUF, False)
        for ct in range(n - NBUF, n):
            wait_scatter(ct % NBUF)

        plsc.subcore_barrier()
        pltpu.sync_copy(accum.at[pl.ds(s * rows_per_tile, rows_per_tile)],
                        out_h.at[c, pl.ds(s * rows_per_tile, rows_per_tile)])

    return sc_kernel(edata, z)


def _wcat(bases, coeff):
    din, dout = bases.shape[1], bases.shape[2]
    return jnp.einsum("rb,bio->iro", coeff, bases).reshape(din, NUM_RELS * dout)


def kernel(feats, edge_index, etype, norm, bases0, coeff0, bias0,
           bases1, coeff1, bias1, bases2, coeff2, bias2):
    src = edge_index[0].astype(jnp.int32)
    dst = edge_index[1].astype(jnp.int32)
    et = etype.astype(jnp.int32)
    nbits = lax.bitcast_convert_type(norm.reshape(-1), jnp.int32)

    edata = jnp.stack([src, et, dst, nbits], axis=0)
    edata = edata.reshape(4, N_CHUNKS, K).transpose(1, 0, 2)

    feats_p = jnp.pad(feats, ((0, NP - N_NODES), (0, 0)))

    z0 = _mm0(feats_p, _wcat(bases0, coeff0))
    p0 = _sc_layer(edata, z0, 128)
    z1 = _mm_fused(p0, bias0, _wcat(bases1, coeff1))
    p1 = _sc_layer(edata, z1, 128)
    z2 = _mm_fused(p1, bias1, _wcat(bases2, coeff2), rel_major=False)
    p2 = _sc_layer(edata, z2, 128, block_select=True)
    out = _combine(p2, bias2, N_NODES, block_rows=1000)
    return out


# speedup vs baseline: 18.4864x; 1.1404x over previous
import functools

import jax
import jax.numpy as jnp
from jax import lax
from jax.experimental import pallas as pl
from jax.experimental.pallas import tpu as pltpu
from jax.experimental.pallas import tpu_sc as plsc

N_NODES = 10000
NP = 10240
N_EDGES = 320000
NUM_RELS = 8
K = 80
NW = 32
N_CHUNKS = N_EDGES // K
CHUNKS_PER_TILE = N_CHUNKS // NW


def _mm0_body(x_ref, w_ref, o_ref):
    o_ref[...] = jnp.dot(x_ref[...], w_ref[...],
                         preferred_element_type=jnp.float32)


def _mm0(x, w, block_rows=1024):
    n, k = x.shape
    _, m = w.shape
    dout = m // NUM_RELS
    nb = n // block_rows
    return pl.pallas_call(
        _mm0_body,
        grid=(nb, NUM_RELS),
        in_specs=[
            pl.BlockSpec((block_rows, k), lambda i, r: (i, 0)),
            pl.BlockSpec((k, dout), lambda i, r: (0, r)),
        ],
        out_specs=pl.BlockSpec((block_rows, dout), lambda i, r: (r * nb + i, 0)),
        out_shape=jax.ShapeDtypeStruct((NUM_RELS * n, dout), jnp.float32),
    )(x, w)


def _mm_fused_body(p_ref, b_ref, w_ref, o_ref):
    x = jax.nn.relu(p_ref[0] + p_ref[1] + b_ref[...])
    o_ref[...] = jnp.dot(x, w_ref[...], preferred_element_type=jnp.float32)


def _mm_fused(partials, bias, w, block_rows=1024, rel_major=True):
    _, n, k = partials.shape
    _, m = w.shape
    if not rel_major:
        return pl.pallas_call(
            _mm_fused_body,
            grid=(n // block_rows,),
            in_specs=[
                pl.BlockSpec((2, block_rows, k), lambda i: (0, i, 0)),
                pl.BlockSpec((1, k), lambda i: (0, 0)),
                pl.BlockSpec((k, m), lambda i: (0, 0)),
            ],
            out_specs=pl.BlockSpec((block_rows, m), lambda i: (i, 0)),
            out_shape=jax.ShapeDtypeStruct((n, m), jnp.float32),
        )(partials, bias.reshape(1, k), w)
    dout = m // NUM_RELS
    nb = n // block_rows
    return pl.pallas_call(
        _mm_fused_body,
        grid=(nb, NUM_RELS),
        in_specs=[
            pl.BlockSpec((2, block_rows, k), lambda i, r: (0, i, 0)),
            pl.BlockSpec((1, k), lambda i, r: (0, 0)),
            pl.BlockSpec((k, dout), lambda i, r: (0, r)),
        ],
        out_specs=pl.BlockSpec((block_rows, dout), lambda i, r: (r * nb + i, 0)),
        out_shape=jax.ShapeDtypeStruct((NUM_RELS * n, dout), jnp.float32),
    )(partials, bias.reshape(1, k), w)


def _combine_body(p_ref, b_ref, o_ref):
    acc = b_ref[...]
    s = p_ref[0] + p_ref[1]
    for d in range(8):
        acc = acc + s[:, d * 16:(d + 1) * 16]
    o_ref[...] = acc


def _combine(partials, bias, n_out, block_rows=1000):
    _, n, k = partials.shape
    m = bias.shape[0]
    return pl.pallas_call(
        _combine_body,
        grid=(n_out // block_rows,),
        in_specs=[
            pl.BlockSpec((2, block_rows, k), lambda i: (0, i, 0)),
            pl.BlockSpec((1, m), lambda i: (0, 0)),
        ],
        out_specs=pl.BlockSpec((block_rows, m), lambda i: (i, 0)),
        out_shape=jax.ShapeDtypeStruct((n_out, m), jnp.float32),
    )(partials, bias.reshape(1, m))


NBUF = 3


def _sc_layer(edata, z, dout, block_select=False):
    mesh = plsc.VectorSubcoreMesh(core_axis_name="c", subcore_axis_name="s")
    rows_per_tile = NP // 16

    @functools.partial(
        pl.kernel,
        out_type=jax.ShapeDtypeStruct((2, NP, dout), jnp.float32),
        mesh=mesh,
        scratch_types=(
            [pltpu.VMEM((4, K), jnp.int32) for _ in range(NBUF)]
            + [pltpu.VMEM((1, K), jnp.int32) for _ in range(NBUF)]
            + [pltpu.VMEM((K,), jnp.int32) for _ in range(NBUF)]
            + [pltpu.VMEM((K, dout), jnp.float32) for _ in range(NBUF)]
            + [pltpu.VMEM_SHARED((NP, dout), jnp.float32)]
            + [pltpu.SemaphoreType.DMA for _ in range(3 * NBUF)]
        ),
    )
    def sc_kernel(edata_h, z_h, out_h, *refs):
        mbuf = refs[0:NBUF]
        dstb = refs[NBUF:2 * NBUF]
        idxb = refs[2 * NBUF:3 * NBUF]
        rows = refs[3 * NBUF:4 * NBUF]
        accum = refs[4 * NBUF]
        msem = refs[4 * NBUF + 1:5 * NBUF + 1]
        gsem = refs[5 * NBUF + 1:6 * NBUF + 1]
        ssem = refs[6 * NBUF + 1:7 * NBUF + 1]

        c = lax.axis_index("c")
        s = lax.axis_index("s")
        wid = s * 2 + c
        n = CHUNKS_PER_TILE

        def zrow(r, _):
            for d in range(dout // 16):
                rows[0][r, pl.ds(d * 16, 16)] = jnp.zeros((16,), jnp.float32)
            return 0
        lax.fori_loop(0, K, zrow, 0)
        for j in range(rows_per_tile // K):
            pltpu.sync_copy(rows[0],
                            accum.at[pl.ds(s * rows_per_tile + j * K, K)])
        plsc.subcore_barrier()

        def issue_meta(i, p):
            pltpu.async_copy(edata_h.at[wid + i * NW], mbuf[p], msem[p])

        def issue_meta_guarded(i, p):
            @pl.when(i < n)
            def _():
                issue_meta(i, p)

        def wait_meta(i, p):
            pltpu.make_async_copy(edata_h.at[wid + i * NW], mbuf[p],
                                  msem[p]).wait()

        def issue_gather(p):
            pltpu.async_copy(z_h.at[idxb[p]], rows[p], gsem[p])

        def wait_gather(p):
            pltpu.make_async_copy(z_h.at[idxb[p]], rows[p], gsem[p]).wait()

        def issue_scatter(p):
            pltpu.async_copy(rows[p], accum.at[dstb[p].at[0]], ssem[p],
                             add=True)

        def wait_scatter(p):
            pltpu.make_async_copy(rows[p], accum.at[dstb[p].at[0]],
                                  ssem[p]).wait()

        def scale_rows(p):
            def scale(g, _):
                nv = lax.bitcast_convert_type(mbuf[p][3, pl.ds(g * 16, 16)],
                                              jnp.float32)
                if block_select:
                    ev = mbuf[p][1, pl.ds(g * 16, 16)]
                for l in range(16):
                    e = g * 16 + l
                    n_e = nv[l]
                    for d in range(dout // 16):
                        sl = pl.ds(d * 16, 16)
                        if block_select:
                            f = jnp.where(ev[l] == d, n_e, 0.0)
                        else:
                            f = n_e
                        rows[p][e, sl] = rows[p][e, sl] * f
                return 0
            lax.fori_loop(0, K // 16, scale, 0)

        def pre(i, p, first):
            if not first:
                wait_scatter(p)
            wait_meta(i, p)
            for j in range(K // 16):
                sl = pl.ds(j * 16, 16)
                if block_select:
                    idxb[p][sl] = mbuf[p][0, sl]
                else:
                    idxb[p][sl] = mbuf[p][1, sl] * NP + mbuf[p][0, sl]
            issue_gather(p)
            issue_meta_guarded(i + 1, (p + 1) % NBUF)

        def post(i, p):
            wait_gather(p)
            scale_rows(p)
            for j in range(K // 16):
                sl = pl.ds(j * 16, 16)
                dstb[p][0, sl] = mbuf[p][2, sl]
            issue_scatter(p)

        n_steps = (n - 2) // NBUF

        def superstep(si, _, first_sup=False):
            cbase = si * NBUF
            for b in range(NBUF):
                post(cbase + b, b)
                pre(cbase + b + 2, (b + 2) % NBUF, first_sup and b == 0)
            return 0

        issue_meta(0, 0)
        pre(0, 0, True)
        pre(1, 1, True)
        superstep(0, 0, first_sup=True)
        lax.fori_loop(1, n_steps, superstep, 0)

        for t in range(n - NBUF * n_steps):
            ct = NBUF * n_steps + t
            post(ct, ct % NBUF)
            if ct + 2 < n:
                pre(ct + 2, (ct + 2) % NBUF, False)
        for ct in range(n - NBUF, n):
            wait_scatter(ct % NBUF)

        plsc.subcore_barrier()
        pltpu.sync_copy(accum.at[pl.ds(s * rows_per_tile, rows_per_tile)],
                        out_h.at[c, pl.ds(s * rows_per_tile, rows_per_tile)])

    return sc_kernel(edata, z)


def _wcat(bases, coeff):
    din, dout = bases.shape[1], bases.shape[2]
    return jnp.einsum("rb,bio->iro", coeff, bases).reshape(din, NUM_RELS * dout)


def kernel(feats, edge_index, etype, norm, bases0, coeff0, bias0,
           bases1, coeff1, bias1, bases2, coeff2, bias2):
    src = edge_index[0].astype(jnp.int32)
    dst = edge_index[1].astype(jnp.int32)
    et = etype.astype(jnp.int32)
    nbits = lax.bitcast_convert_type(norm.reshape(-1), jnp.int32)

    edata = jnp.stack([src, et, dst, nbits], axis=0)
    edata = edata.reshape(4, N_CHUNKS, K).transpose(1, 0, 2)

    feats_p = jnp.pad(feats, ((0, NP - N_NODES), (0, 0)))

    z0 = _mm0(feats_p, _wcat(bases0, coeff0))
    p0 = _sc_layer(edata, z0, 128)
    z1 = _mm_fused(p0, bias0, _wcat(bases1, coeff1))
    p1 = _sc_layer(edata, z1, 128)
    z2 = _mm_fused(p1, bias1, _wcat(bases2, coeff2), rel_major=False)
    p2 = _sc_layer(edata, z2, 128, block_select=True)
    out = _combine(p2, bias2, N_NODES, block_rows=1000)
    return out
